# Initial kernel scaffold; baseline (speedup 1.0000x reference)
import jax
import jax.numpy as jnp
from jax.experimental import pallas as pl


def kernel(history_seq, candidate_ids, category_ids, params):
    raise NotImplementedError("write your pallas kernel here")


# speedup vs baseline: 2.4149x; 2.4149x over previous
import functools
import math

import jax
import jax.numpy as jnp
from jax import lax
from jax.experimental import pallas as pl
from jax.experimental.pallas import tpu as pltpu
from jax.experimental.pallas import tpu_sc as plsc

B, S, NCAND = 1024, 50, 32
D, NH, NE, TOPK, NL = 128, 4, 4, 2, 2
VOCAB, NCAT, HID = 100000, 100, 256
DH = D // NH
SP = 56
MB = 4
MBI = 8
NTOK_H = B * S
NTOK_I = B * NCAND

_INV_SQRT_DH = 1.0 / math.sqrt(DH)
_NEG = -1e30


def _mm_nt(a, b):
    return lax.dot_general(a, b, (((1,), (1,)), ((), ())),
                           preferred_element_type=jnp.float32)


def _mm_nn(a, b):
    return lax.dot_general(a, b, (((1,), (0,)), ((), ())),
                           preferred_element_type=jnp.float32)


def _layernorm(x, g, b):
    m = jnp.mean(x, axis=-1, keepdims=True)
    xc = x - m
    v = jnp.mean(xc * xc, axis=-1, keepdims=True)
    return xc / jnp.sqrt(v + 1e-5) * g + b


def _gelu(x):
    return 0.5 * x * (1.0 + lax.erf(x * (1.0 / math.sqrt(2.0))))


def _softmax_rows(x):
    m = jnp.max(x, axis=-1, keepdims=True)
    e = jnp.exp(x - m)
    return e / jnp.sum(e, axis=-1, keepdims=True)


def _top2_comb(w):
    lane = lax.broadcasted_iota(jnp.int32, w.shape, 1)
    rank = jnp.zeros(w.shape, jnp.int32)
    for j in range(NE):
        wj = w[:, j:j + 1]
        rank = rank + (wj > w).astype(jnp.int32) \
                    + ((wj == w) & (lane > j)).astype(jnp.int32)
    sel = rank < TOPK
    comb = jnp.where(sel, w, 0.0)
    return comb / jnp.sum(comb, axis=-1, keepdims=True)


def _moe_block(xf, gw, gb, w1, b1, w2, b2):
    logits = _mm_nt(xf, gw) + gb
    probs = _softmax_rows(logits)
    comb = _top2_comb(probs)
    acc = jnp.zeros(xf.shape, jnp.float32)
    for e in range(NE):
        h = _gelu(_mm_nt(xf, w1[e]) + b1[e])
        o = _mm_nt(h, w2[e]) + b2[e]
        acc = acc + o * comb[:, e:e + 1]
    return acc, probs


def _attention(q, k, v, mask, nrows_kv):
    del nrows_kv
    outs = []
    for h in range(NH):
        qh = q[:, h * DH:(h + 1) * DH]
        kh = k[:, h * DH:(h + 1) * DH]
        vh = v[:, h * DH:(h + 1) * DH]
        sc = _mm_nt(qh, kh) * _INV_SQRT_DH
        sc = jnp.where(mask, sc, _NEG)
        outs.append(_mm_nn(_softmax_rows(sc), vh))
    return jnp.concatenate(outs, axis=-1)


def _gather_rows(table, idx):
    n_rows = idx.shape[0]
    info = plsc.get_sparse_core_info()
    nw = info.num_cores * info.num_subcores
    b_per_w = n_rows // nw
    chunk = 128
    nch = b_per_w // chunk
    mesh = plsc.VectorSubcoreMesh(core_axis_name="c", subcore_axis_name="s")

    @functools.partial(
        pl.kernel, mesh=mesh,
        out_type=jax.ShapeDtypeStruct((n_rows, D), jnp.float32),
        scratch_types=[
            pltpu.VMEM((chunk,), jnp.int32),
            pltpu.VMEM((chunk, D), jnp.float32),
            pltpu.SemaphoreType.DMA,
        ],
    )
    def k(table_hbm, idx_hbm, out_hbm, idx_v, rows_v, sem):
        wid = lax.axis_index("s") * info.num_cores + lax.axis_index("c")
        base = wid * b_per_w

        def body(i, carry):
            off = base + i * chunk
            pltpu.sync_copy(idx_hbm.at[pl.ds(off, chunk)], idx_v)
            pltpu.async_copy(table_hbm.at[idx_v], rows_v, sem).wait()
            pltpu.sync_copy(rows_v, out_hbm.at[pl.ds(off, chunk)])
            return carry

        lax.fori_loop(0, nch, body, 0)

    return k(table, idx)


def _hist_kernel(x_ref, cos_ref, sin_ref, qkvw_ref, qkvb_ref, outw_ref,
                 outb_ref, ln1g_ref, ln1b_ref, ln2g_ref, ln2b_ref,
                 gatew_ref, gateb_ref, w1_ref, b1_ref, w2_ref, b2_ref,
                 ung_ref, unb_ref, hist_ref, rsum_ref):
    pid = pl.program_id(0)
    T = MB * SP
    x = x_ref[...].reshape(T, D)

    cos = jnp.concatenate([cos_ref[...]] * MB, axis=0)
    sin = jnp.concatenate([sin_ref[...]] * MB, axis=0)
    xr = jnp.concatenate([x[:, 1:], x[:, :1]], axis=-1)
    x = x * cos + xr * sin

    qi = lax.broadcasted_iota(jnp.int32, (T, T), 0)
    ki = lax.broadcasted_iota(jnp.int32, (T, T), 1)
    attn_mask = ((qi // SP) == (ki // SP)) & ((ki % SP) < S)
    tok_pos = lax.broadcasted_iota(jnp.int32, (T, 1), 0) % SP
    real_tok = tok_pos < S

    @pl.when(pid == 0)
    def _():
        rsum_ref[...] = jnp.zeros_like(rsum_ref)

    for l in range(NL):
        qkv = _mm_nt(x, qkvw_ref[l]) + qkvb_ref[l]
        a = _attention(qkv[:, :D], qkv[:, D:2 * D], qkv[:, 2 * D:],
                       attn_mask, T)
        a = _mm_nt(a, outw_ref[l]) + outb_ref[l]
        x = _layernorm(x + a, ln1g_ref[l], ln1b_ref[l])
        routed, probs = _moe_block(x, gatew_ref[l], gateb_ref[l],
                                   w1_ref[l], b1_ref[l], w2_ref[l], b2_ref[l])
        part = jnp.sum(jnp.where(real_tok, probs, 0.0), axis=0, keepdims=True)
        rsum_ref[l:l + 1, :] += part
        x = _layernorm(x + routed, ln2g_ref[l], ln2b_ref[l])

    x = _layernorm(x, ung_ref[...], unb_ref[...])
    hist_ref[...] = x.reshape(MB, SP, D)


def _item_kernel(hist_ref, cand_ref, catid_ref, catemb_ref, gatew_ref,
                 gateb_ref, w1_ref, b1_ref, w2_ref, b2_ref, ing_ref, inb_ref,
                 caw_ref, cab_ref, caow_ref, caob_ref, fw1_ref, fb1_ref,
                 fw2_ref, fb2_ref, rsumh_ref, logit_ref, aux_ref,
                 acc_ref):
    pid = pl.program_id(0)
    nsteps = pl.num_programs(0)
    T = MBI * NCAND
    TK = MBI * SP

    @pl.when(pid == 0)
    def _():
        acc_ref[...] = jnp.zeros_like(acc_ref)

    ids = catid_ref[...]
    onehot = (lax.broadcasted_iota(jnp.int32, (T, 128), 1) == ids)
    cat = lax.dot_general(onehot.astype(jnp.float32), catemb_ref[...],
                          (((1,), (0,)), ((), ())),
                          precision=lax.Precision.HIGHEST,
                          preferred_element_type=jnp.float32)

    item0 = cand_ref[...].reshape(T, D) + cat
    routed, probs = _moe_block(item0, gatew_ref[...], gateb_ref[...],
                               w1_ref[...], b1_ref[...], w2_ref[...],
                               b2_ref[...])
    acc_ref[0:1, 0:NE] += jnp.sum(probs, axis=0, keepdims=True)
    item = _layernorm(routed, ing_ref[...], inb_ref[...])

    histf = hist_ref[...].reshape(TK, D)
    q = _mm_nt(item, caw_ref[0:D]) + cab_ref[:, 0:D]
    k = _mm_nt(histf, caw_ref[D:2 * D]) + cab_ref[:, D:2 * D]
    v = _mm_nt(histf, caw_ref[2 * D:]) + cab_ref[:, 2 * D:]

    qi = lax.broadcasted_iota(jnp.int32, (T, TK), 0)
    ki = lax.broadcasted_iota(jnp.int32, (T, TK), 1)
    mask = ((qi // NCAND) == (ki // SP)) & ((ki % SP) < S)
    ta = _attention(q, k, v, mask, TK)
    ta = _mm_nt(ta, caow_ref[...]) + caob_ref[...]

    fused = jnp.concatenate([ta, item], axis=-1)
    h = _gelu(_mm_nt(fused, fw1_ref[...]) + fb1_ref[...])
    logit_ref[...] = jnp.sum(h * fw2_ref[...], axis=-1, keepdims=True) \
        + fb2_ref[...]

    @pl.when(pid == nsteps - 1)
    def _():
        aux = jnp.zeros((1, 1), jnp.float32)
        for l in range(NL):
            avg = rsumh_ref[l:l + 1, :] * (1.0 / NTOK_H)
            aux += jnp.sum((avg - 1.0 / NE) ** 2, axis=-1, keepdims=True) / NE
        avg_i = acc_ref[0:1, 0:NE] * (1.0 / NTOK_I)
        aux += jnp.sum((avg_i - 1.0 / NE) ** 2, axis=-1, keepdims=True) / NE
        aux_ref[...] = aux


def _full_spec(shape):
    return pl.BlockSpec(shape, lambda i: tuple(0 for _ in shape))


def _hist_tower(xh, p):
    cos_t, sin_t = _rope_tables()
    qkvw = jnp.stack([p['qkv_w%d' % i] for i in range(NL)])
    qkvb = jnp.stack([p['qkv_b%d' % i].reshape(1, 3 * D) for i in range(NL)])
    outw = jnp.stack([p['out_w%d' % i] for i in range(NL)])
    outb = jnp.stack([p['out_b%d' % i].reshape(1, D) for i in range(NL)])
    ln1g = jnp.stack([p['ln1_g%d' % i].reshape(1, D) for i in range(NL)])
    ln1b = jnp.stack([p['ln1_b%d' % i].reshape(1, D) for i in range(NL)])
    ln2g = jnp.stack([p['ln2_g%d' % i].reshape(1, D) for i in range(NL)])
    ln2b = jnp.stack([p['ln2_b%d' % i].reshape(1, D) for i in range(NL)])
    gatew = jnp.stack([p['gate_w%d' % i] for i in range(NL)])
    gateb = jnp.stack([p['gate_b%d' % i].reshape(1, NE) for i in range(NL)])
    w1 = jnp.stack([p['w1_%d' % i] for i in range(NL)])
    b1 = jnp.stack([p['b1_%d' % i].reshape(NE, 1, HID) for i in range(NL)])
    w2 = jnp.stack([p['w2_%d' % i] for i in range(NL)])
    b2 = jnp.stack([p['b2_%d' % i].reshape(NE, 1, D) for i in range(NL)])

    full = _full_spec
    hist, rsum = pl.pallas_call(
        _hist_kernel,
        grid=(B // MB,),
        in_specs=[
            pl.BlockSpec((MB, SP, D), lambda i: (i, 0, 0)),
            full((SP, D)), full((SP, D)),
            full((NL, 3 * D, D)), full((NL, 1, 3 * D)),
            full((NL, D, D)), full((NL, 1, D)),
            full((NL, 1, D)), full((NL, 1, D)),
            full((NL, 1, D)), full((NL, 1, D)),
            full((NL, NE, D)), full((NL, 1, NE)),
            full((NL, NE, HID, D)), full((NL, NE, 1, HID)),
            full((NL, NE, D, HID)), full((NL, NE, 1, D)),
            full((1, D)), full((1, D)),
        ],
        out_specs=[
            pl.BlockSpec((MB, SP, D), lambda i: (i, 0, 0)),
            pl.BlockSpec((8, NE), lambda i: (0, 0)),
        ],
        out_shape=[
            jax.ShapeDtypeStruct((B, SP, D), jnp.float32),
            jax.ShapeDtypeStruct((8, NE), jnp.float32),
        ],
    )(xh, cos_t, sin_t, qkvw, qkvb, outw, outb, ln1g, ln1b, ln2g, ln2b,
      gatew, gateb, w1, b1, w2, b2,
      p['un_g'].reshape(1, D), p['un_b'].reshape(1, D))
    return hist, rsum


def _item_stage(hist, cand, category_ids, p, rsum):
    full = _full_spec
    catemb = jnp.zeros((128, D), jnp.float32).at[:NCAT].set(p['cat_emb'])
    logits, aux = pl.pallas_call(
        _item_kernel,
        grid=(B // MBI,),
        in_specs=[
            pl.BlockSpec((MBI, SP, D), lambda i: (i, 0, 0)),
            pl.BlockSpec((MBI, NCAND, D), lambda i: (i, 0, 0)),
            pl.BlockSpec((MBI * NCAND, 1), lambda i: (i, 0)),
            full((128, D)),
            full((NE, D)), full((1, NE)),
            full((NE, HID, D)), full((NE, 1, HID)),
            full((NE, D, HID)), full((NE, 1, D)),
            full((1, D)), full((1, D)),
            full((3 * D, D)), full((1, 3 * D)),
            full((D, D)), full((1, D)),
            full((D, 2 * D)), full((1, D)),
            full((1, D)), full((1, 1)),
            full((8, NE)),
        ],
        out_specs=[
            pl.BlockSpec((MBI * NCAND, 1), lambda i: (i, 0)),
            pl.BlockSpec((1, 1), lambda i: (0, 0)),
        ],
        out_shape=[
            jax.ShapeDtypeStruct((B * NCAND, 1), jnp.float32),
            jax.ShapeDtypeStruct((1, 1), jnp.float32),
        ],
        scratch_shapes=[pltpu.VMEM((8, 128), jnp.float32)],
    )(hist, cand, category_ids.astype(jnp.int32).reshape(B * NCAND, 1), catemb,
      p['it_gate_w'], p['it_gate_b'].reshape(1, NE),
      p['it_w1'], p['it_b1'].reshape(NE, 1, HID),
      p['it_w2'], p['it_b2'].reshape(NE, 1, D),
      p['in_g'].reshape(1, D), p['in_b'].reshape(1, D),
      p['ca_qkv_w'], p['ca_qkv_b'].reshape(1, 3 * D),
      p['ca_out_w'], p['ca_out_b'].reshape(1, D),
      p['fus_w1'], p['fus_b1'].reshape(1, D),
      p['fus_w2'], p['fus_b2'].reshape(1, 1),
      rsum)
    return logits.reshape(B, NCAND), aux.reshape(())


def _run_towers(xh, cand, category_ids, p):
    hist, rsum = _hist_tower(xh, p)
    return _item_stage(hist, cand, category_ids, p, rsum)


def _rope_tables():
    inv = 1.0 / 10000 ** (jnp.arange(0, D, 2, dtype=jnp.float32) / D)
    t = jnp.arange(SP, dtype=jnp.float32)
    fr = jnp.einsum('i,j->ij', t, inv)
    emb = jnp.concatenate([fr, fr], axis=-1)
    return jnp.cos(emb), jnp.sin(emb)


def kernel(history_seq, candidate_ids, category_ids, params):
    p = params
    hist_idx = jnp.pad(history_seq.astype(jnp.int32),
                       ((0, 0), (0, SP - S))).reshape(-1)
    cand_idx = candidate_ids.astype(jnp.int32).reshape(-1)
    xh = _gather_rows(p['item_emb'], hist_idx).reshape(B, SP, D)
    cand = _gather_rows(p['item_emb'], cand_idx).reshape(B, NCAND, D)
    return _run_towers(xh, cand, category_ids, p)
